# baseline (speedup 1.0000x reference)
import functools

import jax
import jax.numpy as jnp
from jax import lax
from jax.experimental import pallas as pl
from jax.experimental.pallas import tpu as pltpu
from jax.experimental.pallas import tpu_sc as plsc

F32 = jnp.float32

_N = 10000
_E = 320000
_DN = 128
_DE = 16
_DG = 8

_CHUNK = 128
_K = 4
_GROUP = _K * _CHUNK
_NGROUPS = _E // _GROUP
_NW = 32
_NJ = 20
_NGPAD = _NJ * _NW
_EPAD = _NGPAD * _GROUP
_NCHPAD = _EPAD // _CHUNK
_NPAD = 10240
_ROWS_PER_SUB = _NPAD // 16

_NBLK = 400
_E8 = _E // 8
_EBLK8 = 2000


def _init_body(nodes_ref, g_ref, wes_ref, wer_ref, weg_ref, web_ref,
               p_ref, q_ref, ce_ref):
    nb = nodes_ref[...]
    p_ref[...] = jnp.dot(nb, wes_ref[...], preferred_element_type=F32)
    q_ref[...] = jnp.dot(nb, wer_ref[...], preferred_element_type=F32)

    @pl.when(pl.program_id(0) == 0)
    def _():
        ce_ref[...] = (
            jnp.dot(g_ref[...], weg_ref[...], preferred_element_type=F32)
            + web_ref[...]
        )


def _edge_pre_body(edges_ref, wbd_ref, cet_ref, ee_ref):
    ee_ref[...] = (
        jnp.dot(edges_ref[...], wbd_ref[...], preferred_element_type=F32)
        + cet_ref[...]
    )


def _sc_step_body(ee_hbm, p_hbm, q_hbm, sr_hbm,
                  eout_hbm, recv_hbm,
                  idx_v, ee_v, ps_v, qs_v, out_v, acc_sh,
                  esem0, esem1, gsem0, gsem1, isem0, isem1):
    cid = lax.axis_index("c")
    sid = lax.axis_index("s")
    wid = sid * 2 + cid
    row0 = sid * _ROWS_PER_SUB
    esem = (esem0, esem1)
    gsem = (gsem0, gsem1)
    isem = (isem0, isem1)

    def issue_idx(j, b):
        grp = wid + j * _NW
        pltpu.async_copy(sr_hbm.at[pl.ds(grp * _K, _K)], idx_v.at[b], isem[b])

    def wait_idx(j, b):
        grp = wid + j * _NW
        pltpu.make_async_copy(
            sr_hbm.at[pl.ds(grp * _K, _K)], idx_v.at[b], isem[b]).wait()

    def main_copies(j, b):
        grp = wid + j * _NW
        base = grp * _GROUP
        yield ee_hbm.at[pl.ds(base, _GROUP)], ee_v.at[b], esem[b]
        for k in range(_K):
            sl = pl.ds(k * _CHUNK, _CHUNK)
            yield p_hbm.at[idx_v.at[b, k, 0]], ps_v.at[b].at[sl], gsem[b]
            yield q_hbm.at[idx_v.at[b, k, 1]], qs_v.at[b].at[sl], gsem[b]

    def issue_main(j, b):
        for src, dst, sm in main_copies(j, b):
            pltpu.async_copy(src, dst, sm)

    def wait_main(j, b):
        for src, dst, sm in main_copies(j, b):
            pltpu.make_async_copy(src, dst, sm).wait()

    def finish(j, b):
        grp = wid + j * _NW
        base = grp * _GROUP
        eb, pb, qb, ob = ee_v.at[b], ps_v.at[b], qs_v.at[b], out_v.at[b]

        @pl.when(grp < _NGROUPS)
        def _():
            def row_body(i, a):
                r = i * 4
                for u in range(4):
                    ob[r + u, :] = jnp.maximum(
                        eb[r + u, :] + pb[r + u, :] + qb[r + u, :], 0.0)
                return a
            lax.fori_loop(0, _GROUP // 4, row_body, 0)

        @pl.when(grp >= _NGROUPS)
        def _():
            zrow = jnp.zeros((_DE,), F32)
            def row_body0(i, a):
                r = i * 4
                for u in range(4):
                    ob[r + u, :] = zrow
                return a
            lax.fori_loop(0, _GROUP // 4, row_body0, 0)

        pltpu.sync_copy(ob, eout_hbm.at[pl.ds(base, _GROUP)])
        for k in range(_K):
            pltpu.sync_copy(ob.at[pl.ds(k * _CHUNK, _CHUNK)],
                            acc_sh.at[idx_v.at[b, k, 1]], add=True)

    pltpu.sync_copy(sr_hbm.at[pl.ds(wid * _K, _K)], idx_v.at[0])
    issue_main(0, 0)
    issue_idx(1, 1)

    zrow = jnp.zeros((_DE,), F32)
    def zero_body(i, a):
        r = i * 8
        for u in range(8):
            out_v[0, r + u, :] = zrow
        return a
    lax.fori_loop(0, _GROUP // 8, zero_body, 0)
    pltpu.sync_copy(out_v.at[0], acc_sh.at[pl.ds(row0, _GROUP)])
    pltpu.sync_copy(out_v.at[0].at[pl.ds(0, _ROWS_PER_SUB - _GROUP)],
                    acc_sh.at[pl.ds(row0 + _GROUP, _ROWS_PER_SUB - _GROUP)])
    plsc.subcore_barrier()

    def steady(jj, carry):
        for b in (0, 1):
            j = 2 * jj + b
            wait_idx(j + 1, 1 - b)
            wait_main(j, b)
            issue_main(j + 1, 1 - b)
            finish(j, b)
            issue_idx(j + 2, b)
        return carry

    lax.fori_loop(0, _NJ // 2 - 1, steady, 0)

    wait_idx(_NJ - 1, 1)
    wait_main(_NJ - 2, 0)
    issue_main(_NJ - 1, 1)
    finish(_NJ - 2, 0)
    wait_main(_NJ - 1, 1)
    finish(_NJ - 1, 1)

    plsc.subcore_barrier()
    pltpu.sync_copy(acc_sh.at[pl.ds(row0, _ROWS_PER_SUB)],
                    recv_hbm.at[cid, pl.ds(row0, _ROWS_PER_SUB)])


def _node_body(nodes_ref, recv2_ref, g_ref,
               wnn_ref, wnr_ref, wng_ref, wnb_ref,
               wes_ref, wer_ref, weg_ref, web_ref,
               wgn_ref, wge_ref, wgg_ref, wgb_ref,
               nout_ref, p_ref, q_ref, gout_ref, ce_ref,
               accn_ref, acce_ref):
    i = pl.program_id(0)
    g = g_ref[...]
    c_n = jnp.dot(g, wng_ref[...], preferred_element_type=F32) + wnb_ref[...]
    recv = recv2_ref[0] + recv2_ref[1]
    out = (
        jnp.dot(nodes_ref[...], wnn_ref[...], preferred_element_type=F32)
        + jnp.dot(recv, wnr_ref[...], preferred_element_type=F32)
        + c_n
    )
    out = jnp.maximum(out, 0.0)
    nout_ref[...] = out
    p_ref[...] = jnp.dot(out, wes_ref[...], preferred_element_type=F32)
    q_ref[...] = jnp.dot(out, wer_ref[...], preferred_element_type=F32)

    @pl.when(i == 0)
    def _():
        accn_ref[...] = jnp.zeros_like(accn_ref)
        acce_ref[...] = jnp.zeros_like(acce_ref)

    accn_ref[...] += jnp.sum(out, axis=0, keepdims=True)
    acce_ref[...] += jnp.sum(recv, axis=0, keepdims=True)

    @pl.when(i == pl.num_programs(0) - 1)
    def _():
        agg_n = accn_ref[...]
        agg_e = acce_ref[...]
        g_new = (
            jnp.dot(agg_n, wgn_ref[...], preferred_element_type=F32)
            + jnp.dot(agg_e, wge_ref[...], preferred_element_type=F32)
            + jnp.dot(g, wgg_ref[...], preferred_element_type=F32)
            + wgb_ref[...]
        )
        gout_ref[...] = g_new
        ce_ref[...] = (
            jnp.dot(g_new, weg_ref[...], preferred_element_type=F32)
            + web_ref[...]
        )


def _full(i):
    return 0


def kernel(nodes, edges, globals_, senders, receivers,
           We_W, We_b, Wn_W, Wn_b, Wg_W, Wg_b):
    We_e = We_W[:_DE]
    We_s = We_W[_DE:_DE + _DN]
    We_r = We_W[_DE + _DN:_DE + 2 * _DN]
    We_g = We_W[_DE + 2 * _DN:]
    Wn_n = Wn_W[:_DN]
    Wn_r = Wn_W[_DN:_DN + _DE]
    Wn_g = Wn_W[_DN + _DE:]
    Wg_n = Wg_W[:_DN]
    Wg_e = Wg_W[_DN:_DN + _DE]
    Wg_g = Wg_W[_DN + _DE:]
    web = We_b.reshape(1, _DE)
    wnb = Wn_b.reshape(1, _DN)
    wgb = Wg_b.reshape(1, _DG)
    idx_pad = jnp.zeros((_EPAD - _E,), senders.dtype)
    sr_packed = jnp.stack(
        [jnp.concatenate([senders, idx_pad]).reshape(_NCHPAD, _CHUNK),
         jnp.concatenate([receivers, idx_pad]).reshape(_NCHPAD, _CHUNK)],
        axis=1)

    n_grid = _N // _NBLK
    e_grid = _E8 // _EBLK8

    p0, q0, ce0 = pl.pallas_call(
        _init_body,
        grid=(n_grid,),
        in_specs=[
            pl.BlockSpec((_NBLK, _DN), lambda i: (i, 0)),
            pl.BlockSpec((1, _DG), lambda i: (0, 0)),
            pl.BlockSpec((_DN, _DE), lambda i: (0, 0)),
            pl.BlockSpec((_DN, _DE), lambda i: (0, 0)),
            pl.BlockSpec((_DG, _DE), lambda i: (0, 0)),
            pl.BlockSpec((1, _DE), lambda i: (0, 0)),
        ],
        out_specs=[
            pl.BlockSpec((_NBLK, _DE), lambda i: (i, 0)),
            pl.BlockSpec((_NBLK, _DE), lambda i: (i, 0)),
            pl.BlockSpec((1, _DE), lambda i: (0, 0)),
        ],
        out_shape=[
            jax.ShapeDtypeStruct((_N, _DE), F32),
            jax.ShapeDtypeStruct((_N, _DE), F32),
            jax.ShapeDtypeStruct((1, _DE), F32),
        ],
    )(nodes, globals_, We_s, We_r, We_g, web)

    edge_pre = pl.pallas_call(
        _edge_pre_body,
        grid=(e_grid,),
        in_specs=[
            pl.BlockSpec((_EBLK8, 128), lambda i: (i, 0)),
            pl.BlockSpec((128, 128), lambda i: (0, 0)),
            pl.BlockSpec((1, 128), lambda i: (0, 0)),
        ],
        out_specs=pl.BlockSpec((_EBLK8, 128), lambda i: (i, 0)),
        out_shape=jax.ShapeDtypeStruct((_EPAD // 8, 128), F32),
    )

    sc_step = pl.kernel(
        _sc_step_body,
        out_type=[
            jax.ShapeDtypeStruct((_EPAD, _DE), F32),
            jax.ShapeDtypeStruct((2, _NPAD, _DE), F32),
        ],
        mesh=plsc.VectorSubcoreMesh(core_axis_name="c", subcore_axis_name="s"),
        compiler_params=pltpu.CompilerParams(use_tc_tiling_on_sc=False),
        scratch_types=[
            pltpu.VMEM((2, _K, 2, _CHUNK), jnp.int32),
            pltpu.VMEM((2, _GROUP, _DE), F32),
            pltpu.VMEM((2, _GROUP, _DE), F32),
            pltpu.VMEM((2, _GROUP, _DE), F32),
            pltpu.VMEM((2, _GROUP, _DE), F32),
            pltpu.VMEM_SHARED((_NPAD, _DE), F32),
            pltpu.SemaphoreType.DMA,
            pltpu.SemaphoreType.DMA,
            pltpu.SemaphoreType.DMA,
            pltpu.SemaphoreType.DMA,
            pltpu.SemaphoreType.DMA,
            pltpu.SemaphoreType.DMA,
        ],
    )

    node_step = pl.pallas_call(
        _node_body,
        grid=(n_grid,),
        in_specs=[
            pl.BlockSpec((_NBLK, _DN), lambda i: (i, 0)),
            pl.BlockSpec((2, _NBLK, _DE), lambda i: (0, i, 0)),
            pl.BlockSpec((1, _DG), lambda i: (0, 0)),
            pl.BlockSpec((_DN, _DN), lambda i: (0, 0)),
            pl.BlockSpec((_DE, _DN), lambda i: (0, 0)),
            pl.BlockSpec((_DG, _DN), lambda i: (0, 0)),
            pl.BlockSpec((1, _DN), lambda i: (0, 0)),
            pl.BlockSpec((_DN, _DE), lambda i: (0, 0)),
            pl.BlockSpec((_DN, _DE), lambda i: (0, 0)),
            pl.BlockSpec((_DG, _DE), lambda i: (0, 0)),
            pl.BlockSpec((1, _DE), lambda i: (0, 0)),
            pl.BlockSpec((_DN, _DG), lambda i: (0, 0)),
            pl.BlockSpec((_DE, _DG), lambda i: (0, 0)),
            pl.BlockSpec((_DG, _DG), lambda i: (0, 0)),
            pl.BlockSpec((1, _DG), lambda i: (0, 0)),
        ],
        out_specs=[
            pl.BlockSpec((_NBLK, _DN), lambda i: (i, 0)),
            pl.BlockSpec((_NBLK, _DE), lambda i: (i, 0)),
            pl.BlockSpec((_NBLK, _DE), lambda i: (i, 0)),
            pl.BlockSpec((1, _DG), lambda i: (0, 0)),
            pl.BlockSpec((1, _DE), lambda i: (0, 0)),
        ],
        out_shape=[
            jax.ShapeDtypeStruct((_N, _DN), F32),
            jax.ShapeDtypeStruct((_N, _DE), F32),
            jax.ShapeDtypeStruct((_N, _DE), F32),
            jax.ShapeDtypeStruct((1, _DG), F32),
            jax.ShapeDtypeStruct((1, _DE), F32),
        ],
        scratch_shapes=[pltpu.VMEM((1, _DN), F32), pltpu.VMEM((1, _DE), F32)],
    )

    W_bd = jnp.kron(jnp.eye(8, dtype=F32), We_e)
    edges_p = edges.reshape(_E8, 128)
    p, q, ce, g = p0, q0, ce0, globals_
    for _step in range(3):
        ee_p = edge_pre(edges_p, W_bd, jnp.tile(ce, (1, 8)))
        edges_lin, recv2 = sc_step(
            ee_p.reshape(_EPAD, _DE), p, q, sr_packed)
        edges_p = edges_lin.reshape(_EPAD // 8, 128)
        nodes, p, q, g, ce = node_step(
            nodes, recv2, g,
            Wn_n, Wn_r, Wn_g, wnb,
            We_s, We_r, We_g, web,
            Wg_n, Wg_e, Wg_g, wgb,
        )

    return (nodes, edges_lin[:_E], g)


# speedup vs baseline: 1.1457x; 1.1457x over previous
import functools

import jax
import jax.numpy as jnp
from jax import lax
from jax.experimental import pallas as pl
from jax.experimental.pallas import tpu as pltpu
from jax.experimental.pallas import tpu_sc as plsc

F32 = jnp.float32

_N = 10000
_E = 320000
_DN = 128
_DE = 16
_DG = 8

_CHUNK = 128
_K = 4
_GROUP = _K * _CHUNK
_NGROUPS = _E // _GROUP
_NW = 32
_NJ = 20
_NGPAD = _NJ * _NW
_EPAD = _NGPAD * _GROUP
_NCHPAD = _EPAD // _CHUNK
_NPAD = 10240
_ROWS_PER_SUB = _NPAD // 16

_NBLK = 400
_E8 = _E // 8
_EBLK8 = 2000


def _init_body(nodes_ref, g_ref, wes_ref, wer_ref, weg_ref, web_ref,
               p_ref, q_ref, ce_ref):
    nb = nodes_ref[...]
    p_ref[...] = jnp.dot(nb, wes_ref[...], preferred_element_type=F32)
    q_ref[...] = jnp.dot(nb, wer_ref[...], preferred_element_type=F32)

    @pl.when(pl.program_id(0) == 0)
    def _():
        ce_ref[...] = (
            jnp.dot(g_ref[...], weg_ref[...], preferred_element_type=F32)
            + web_ref[...]
        )


def _edge_pre_body(edges_ref, wbd_ref, cet_ref, ee_ref):
    ee_ref[...] = (
        jnp.dot(edges_ref[...], wbd_ref[...], preferred_element_type=F32)
        + cet_ref[...]
    )


def _sc_step_body(ee_hbm, p_hbm, q_hbm, s_hbm, r_hbm,
                  eout_hbm, trash_hbm, recv_hbm,
                  idxs_v, idxr_v, ee_v, ps_v, qs_v, out_v, acc_sh,
                  esem0, esem1, gsem0, gsem1, isem0, isem1):
    cid = lax.axis_index("c")
    sid = lax.axis_index("s")
    wid = sid * 2 + cid
    row0 = sid * _ROWS_PER_SUB
    esem = (esem0, esem1)
    gsem = (gsem0, gsem1)
    isem = (isem0, isem1)

    def idx_copies(j, b):
        grp = wid + j * _NW
        sl = pl.ds(grp * _K, _K)
        yield s_hbm.at[sl], idxs_v.at[b], isem[b]
        yield r_hbm.at[sl], idxr_v.at[b], isem[b]

    def issue_idx(j, b):
        for src, dst, sm in idx_copies(j, b):
            pltpu.async_copy(src, dst, sm)

    def wait_idx(j, b):
        for src, dst, sm in idx_copies(j, b):
            pltpu.make_async_copy(src, dst, sm).wait()

    def main_copies(j, b):
        grp = wid + j * _NW
        base = grp * _GROUP
        yield ee_hbm.at[pl.ds(base, _GROUP)], ee_v.at[b], esem[b]
        for k in range(_K):
            sl = pl.ds(k * _CHUNK, _CHUNK)
            yield p_hbm.at[idxs_v.at[b, k]], ps_v.at[b].at[sl], gsem[b]
            yield q_hbm.at[idxr_v.at[b, k]], qs_v.at[b].at[sl], gsem[b]

    def issue_main(j, b):
        for src, dst, sm in main_copies(j, b):
            pltpu.async_copy(src, dst, sm)

    def wait_main(j, b):
        for src, dst, sm in main_copies(j, b):
            pltpu.make_async_copy(src, dst, sm).wait()

    def finish(j, b):
        grp = wid + j * _NW
        base = grp * _GROUP
        eb, pb, qb, ob = ee_v.at[b], ps_v.at[b], qs_v.at[b], out_v.at[b]

        @pl.when(grp < _NGROUPS)
        def _():
            def row_body(i, a):
                r = i * 4
                for u in range(4):
                    ob[r + u, :] = jnp.maximum(
                        eb[r + u, :] + pb[r + u, :] + qb[r + u, :], 0.0)
                return a
            lax.fori_loop(0, _GROUP // 4, row_body, 0)

        @pl.when(grp >= _NGROUPS)
        def _():
            zrow = jnp.zeros((_DE,), F32)
            def row_body0(i, a):
                r = i * 4
                for u in range(4):
                    ob[r + u, :] = zrow
                return a
            lax.fori_loop(0, _GROUP // 4, row_body0, 0)

        @pl.when(grp < _NGROUPS)
        def _():
            pltpu.sync_copy(ob, eout_hbm.at[pl.ds(base, _GROUP)])

        @pl.when(grp >= _NGROUPS)
        def _():
            pltpu.sync_copy(ob, trash_hbm)

        for k in range(_K):
            pltpu.sync_copy(ob.at[pl.ds(k * _CHUNK, _CHUNK)],
                            acc_sh.at[idxr_v.at[b, k]], add=True)

    pltpu.sync_copy(s_hbm.at[pl.ds(wid * _K, _K)], idxs_v.at[0])
    pltpu.sync_copy(r_hbm.at[pl.ds(wid * _K, _K)], idxr_v.at[0])
    issue_main(0, 0)
    issue_idx(1, 1)

    zrow = jnp.zeros((_DE,), F32)
    def zero_body(i, a):
        r = i * 8
        for u in range(8):
            out_v[0, r + u, :] = zrow
        return a
    lax.fori_loop(0, _GROUP // 8, zero_body, 0)
    pltpu.sync_copy(out_v.at[0], acc_sh.at[pl.ds(row0, _GROUP)])
    pltpu.sync_copy(out_v.at[0].at[pl.ds(0, _ROWS_PER_SUB - _GROUP)],
                    acc_sh.at[pl.ds(row0 + _GROUP, _ROWS_PER_SUB - _GROUP)])
    plsc.subcore_barrier()

    def steady(jj, carry):
        for b in (0, 1):
            j = 2 * jj + b
            wait_idx(j + 1, 1 - b)
            wait_main(j, b)
            issue_main(j + 1, 1 - b)
            finish(j, b)
            issue_idx(j + 2, b)
        return carry

    lax.fori_loop(0, _NJ // 2 - 1, steady, 0)

    wait_idx(_NJ - 1, 1)
    wait_main(_NJ - 2, 0)
    issue_main(_NJ - 1, 1)
    finish(_NJ - 2, 0)
    wait_main(_NJ - 1, 1)
    finish(_NJ - 1, 1)

    plsc.subcore_barrier()
    pltpu.sync_copy(acc_sh.at[pl.ds(row0, _ROWS_PER_SUB)],
                    recv_hbm.at[cid, pl.ds(row0, _ROWS_PER_SUB)])


def _node_body(nodes_ref, recv2_ref, g_ref,
               wnn_ref, wnr_ref, wng_ref, wnb_ref,
               wes_ref, wer_ref, weg_ref, web_ref,
               wgn_ref, wge_ref, wgg_ref, wgb_ref,
               nout_ref, p_ref, q_ref, gout_ref, ce_ref,
               accn_ref, acce_ref):
    i = pl.program_id(0)
    g = g_ref[...]
    c_n = jnp.dot(g, wng_ref[...], preferred_element_type=F32) + wnb_ref[...]
    recv = recv2_ref[0] + recv2_ref[1]
    out = (
        jnp.dot(nodes_ref[...], wnn_ref[...], preferred_element_type=F32)
        + jnp.dot(recv, wnr_ref[...], preferred_element_type=F32)
        + c_n
    )
    out = jnp.maximum(out, 0.0)
    nout_ref[...] = out
    p_ref[...] = jnp.dot(out, wes_ref[...], preferred_element_type=F32)
    q_ref[...] = jnp.dot(out, wer_ref[...], preferred_element_type=F32)

    @pl.when(i == 0)
    def _():
        accn_ref[...] = jnp.zeros_like(accn_ref)
        acce_ref[...] = jnp.zeros_like(acce_ref)

    accn_ref[...] += jnp.sum(out, axis=0, keepdims=True)
    acce_ref[...] += jnp.sum(recv, axis=0, keepdims=True)

    @pl.when(i == pl.num_programs(0) - 1)
    def _():
        agg_n = accn_ref[...]
        agg_e = acce_ref[...]
        g_new = (
            jnp.dot(agg_n, wgn_ref[...], preferred_element_type=F32)
            + jnp.dot(agg_e, wge_ref[...], preferred_element_type=F32)
            + jnp.dot(g, wgg_ref[...], preferred_element_type=F32)
            + wgb_ref[...]
        )
        gout_ref[...] = g_new
        ce_ref[...] = (
            jnp.dot(g_new, weg_ref[...], preferred_element_type=F32)
            + web_ref[...]
        )


def _full(i):
    return 0


def kernel(nodes, edges, globals_, senders, receivers,
           We_W, We_b, Wn_W, Wn_b, Wg_W, Wg_b):
    We_e = We_W[:_DE]
    We_s = We_W[_DE:_DE + _DN]
    We_r = We_W[_DE + _DN:_DE + 2 * _DN]
    We_g = We_W[_DE + 2 * _DN:]
    Wn_n = Wn_W[:_DN]
    Wn_r = Wn_W[_DN:_DN + _DE]
    Wn_g = Wn_W[_DN + _DE:]
    Wg_n = Wg_W[:_DN]
    Wg_e = Wg_W[_DN:_DN + _DE]
    Wg_g = Wg_W[_DN + _DE:]
    web = We_b.reshape(1, _DE)
    wnb = Wn_b.reshape(1, _DN)
    wgb = Wg_b.reshape(1, _DG)
    idx_pad = jnp.zeros((_EPAD - _E,), senders.dtype)
    spad = jnp.concatenate([senders, idx_pad]).reshape(_NCHPAD, _CHUNK)
    rpad = jnp.concatenate([receivers, idx_pad]).reshape(_NCHPAD, _CHUNK)

    n_grid = _N // _NBLK
    e_grid = _E8 // _EBLK8

    p0, q0, ce0 = pl.pallas_call(
        _init_body,
        grid=(n_grid,),
        in_specs=[
            pl.BlockSpec((_NBLK, _DN), lambda i: (i, 0)),
            pl.BlockSpec((1, _DG), lambda i: (0, 0)),
            pl.BlockSpec((_DN, _DE), lambda i: (0, 0)),
            pl.BlockSpec((_DN, _DE), lambda i: (0, 0)),
            pl.BlockSpec((_DG, _DE), lambda i: (0, 0)),
            pl.BlockSpec((1, _DE), lambda i: (0, 0)),
        ],
        out_specs=[
            pl.BlockSpec((_NBLK, _DE), lambda i: (i, 0)),
            pl.BlockSpec((_NBLK, _DE), lambda i: (i, 0)),
            pl.BlockSpec((1, _DE), lambda i: (0, 0)),
        ],
        out_shape=[
            jax.ShapeDtypeStruct((_N, _DE), F32),
            jax.ShapeDtypeStruct((_N, _DE), F32),
            jax.ShapeDtypeStruct((1, _DE), F32),
        ],
    )(nodes, globals_, We_s, We_r, We_g, web)

    edge_pre = pl.pallas_call(
        _edge_pre_body,
        grid=(e_grid,),
        in_specs=[
            pl.BlockSpec((_EBLK8, 128), lambda i: (i, 0)),
            pl.BlockSpec((128, 128), lambda i: (0, 0)),
            pl.BlockSpec((1, 128), lambda i: (0, 0)),
        ],
        out_specs=pl.BlockSpec((_EBLK8, 128), lambda i: (i, 0)),
        out_shape=jax.ShapeDtypeStruct((_EPAD // 8, 128), F32),
    )

    sc_step = pl.kernel(
        _sc_step_body,
        out_type=[
            jax.ShapeDtypeStruct((_E, _DE), F32),
            jax.ShapeDtypeStruct((_GROUP, _DE), F32),
            jax.ShapeDtypeStruct((2, _NPAD, _DE), F32),
        ],
        mesh=plsc.VectorSubcoreMesh(core_axis_name="c", subcore_axis_name="s"),
        compiler_params=pltpu.CompilerParams(use_tc_tiling_on_sc=False),
        scratch_types=[
            pltpu.VMEM((2, _K, _CHUNK), jnp.int32),
            pltpu.VMEM((2, _K, _CHUNK), jnp.int32),
            pltpu.VMEM((2, _GROUP, _DE), F32),
            pltpu.VMEM((2, _GROUP, _DE), F32),
            pltpu.VMEM((2, _GROUP, _DE), F32),
            pltpu.VMEM((2, _GROUP, _DE), F32),
            pltpu.VMEM_SHARED((_NPAD, _DE), F32),
            pltpu.SemaphoreType.DMA,
            pltpu.SemaphoreType.DMA,
            pltpu.SemaphoreType.DMA,
            pltpu.SemaphoreType.DMA,
            pltpu.SemaphoreType.DMA,
            pltpu.SemaphoreType.DMA,
        ],
    )

    node_step = pl.pallas_call(
        _node_body,
        grid=(n_grid,),
        in_specs=[
            pl.BlockSpec((_NBLK, _DN), lambda i: (i, 0)),
            pl.BlockSpec((2, _NBLK, _DE), lambda i: (0, i, 0)),
            pl.BlockSpec((1, _DG), lambda i: (0, 0)),
            pl.BlockSpec((_DN, _DN), lambda i: (0, 0)),
            pl.BlockSpec((_DE, _DN), lambda i: (0, 0)),
            pl.BlockSpec((_DG, _DN), lambda i: (0, 0)),
            pl.BlockSpec((1, _DN), lambda i: (0, 0)),
            pl.BlockSpec((_DN, _DE), lambda i: (0, 0)),
            pl.BlockSpec((_DN, _DE), lambda i: (0, 0)),
            pl.BlockSpec((_DG, _DE), lambda i: (0, 0)),
            pl.BlockSpec((1, _DE), lambda i: (0, 0)),
            pl.BlockSpec((_DN, _DG), lambda i: (0, 0)),
            pl.BlockSpec((_DE, _DG), lambda i: (0, 0)),
            pl.BlockSpec((_DG, _DG), lambda i: (0, 0)),
            pl.BlockSpec((1, _DG), lambda i: (0, 0)),
        ],
        out_specs=[
            pl.BlockSpec((_NBLK, _DN), lambda i: (i, 0)),
            pl.BlockSpec((_NBLK, _DE), lambda i: (i, 0)),
            pl.BlockSpec((_NBLK, _DE), lambda i: (i, 0)),
            pl.BlockSpec((1, _DG), lambda i: (0, 0)),
            pl.BlockSpec((1, _DE), lambda i: (0, 0)),
        ],
        out_shape=[
            jax.ShapeDtypeStruct((_N, _DN), F32),
            jax.ShapeDtypeStruct((_N, _DE), F32),
            jax.ShapeDtypeStruct((_N, _DE), F32),
            jax.ShapeDtypeStruct((1, _DG), F32),
            jax.ShapeDtypeStruct((1, _DE), F32),
        ],
        scratch_shapes=[pltpu.VMEM((1, _DN), F32), pltpu.VMEM((1, _DE), F32)],
    )

    W_bd = jnp.kron(jnp.eye(8, dtype=F32), We_e)
    edges_p = edges.reshape(_E8, 128)
    p, q, ce, g = p0, q0, ce0, globals_
    for _step in range(3):
        ee_p = edge_pre(edges_p, W_bd, jnp.tile(ce, (1, 8)))
        edges_lin, _trash, recv2 = sc_step(
            ee_p.reshape(_EPAD, _DE), p, q, spad, rpad)
        edges_p = edges_lin.reshape(_E8, 128)
        nodes, p, q, g, ce = node_step(
            nodes, recv2, g,
            Wn_n, Wn_r, Wn_g, wnb,
            We_s, We_r, We_g, web,
            Wg_n, Wg_e, Wg_g, wgb,
        )

    return (nodes, edges_lin, g)


# speedup vs baseline: 1.1895x; 1.0382x over previous
import functools

import jax
import jax.numpy as jnp
from jax import lax
from jax.experimental import pallas as pl
from jax.experimental.pallas import tpu as pltpu
from jax.experimental.pallas import tpu_sc as plsc

F32 = jnp.float32

_N = 10000
_E = 320000
_DN = 128
_DE = 16
_DG = 8

_CHUNK = 128
_K = 4
_GROUP = _K * _CHUNK
_NGROUPS = _E // _GROUP
_NW = 32
_NJ = 20
_NGPAD = _NJ * _NW
_EPAD = _NGPAD * _GROUP
_NCHPAD = _EPAD // _CHUNK
_NPAD = 10240
_ROWS_PER_SUB = _NPAD // 16

_NBLK = 400
_E8 = _E // 8
_EBLKF = _E8 // (_N // _NBLK)


def _init_body(nodes_ref, edges_ref, g_ref, wes_ref, wer_ref, weg_ref,
               web_ref, wbd_ref,
               p_ref, q_ref, ce_ref, ee_ref):
    nb = nodes_ref[...]
    p_ref[...] = jnp.dot(nb, wes_ref[...], preferred_element_type=F32)
    q_ref[...] = jnp.dot(nb, wer_ref[...], preferred_element_type=F32)
    ee_ref[...] = jnp.dot(edges_ref[...], wbd_ref[...],
                          preferred_element_type=F32)

    @pl.when(pl.program_id(0) == 0)
    def _():
        ce_ref[...] = (
            jnp.dot(g_ref[...], weg_ref[...], preferred_element_type=F32)
            + web_ref[...]
        )


def _sc_step_body(ee_hbm, p_hbm, q_hbm, s_hbm, r_hbm, ce_hbm,
                  eout_hbm, trash_hbm, recv_hbm,
                  idxs_v, idxr_v, ee_v, ps_v, qs_v, out_v, cev, acc_sh,
                  esem0, esem1, gsem0, gsem1, isem0, isem1):
    cid = lax.axis_index("c")
    sid = lax.axis_index("s")
    wid = sid * 2 + cid
    row0 = sid * _ROWS_PER_SUB
    esem = (esem0, esem1)
    gsem = (gsem0, gsem1)
    isem = (isem0, isem1)

    def idx_copies(j, b):
        grp = wid + j * _NW
        sl = pl.ds(grp * _K, _K)
        yield s_hbm.at[sl], idxs_v.at[b], isem[b]
        yield r_hbm.at[sl], idxr_v.at[b], isem[b]

    def issue_idx(j, b):
        for src, dst, sm in idx_copies(j, b):
            pltpu.async_copy(src, dst, sm)

    def wait_idx(j, b):
        for src, dst, sm in idx_copies(j, b):
            pltpu.make_async_copy(src, dst, sm).wait()

    def main_copies(j, b):
        grp = wid + j * _NW
        base = grp * _GROUP
        yield ee_hbm.at[pl.ds(base, _GROUP)], ee_v.at[b], esem[b]
        for k in range(_K):
            sl = pl.ds(k * _CHUNK, _CHUNK)
            yield p_hbm.at[idxs_v.at[b, k]], ps_v.at[b].at[sl], gsem[b]
            yield q_hbm.at[idxr_v.at[b, k]], qs_v.at[b].at[sl], gsem[b]

    def issue_main(j, b):
        for src, dst, sm in main_copies(j, b):
            pltpu.async_copy(src, dst, sm)

    def wait_main(j, b):
        for src, dst, sm in main_copies(j, b):
            pltpu.make_async_copy(src, dst, sm).wait()

    def finish(j, b):
        grp = wid + j * _NW
        base = grp * _GROUP
        eb, pb, qb, ob = ee_v.at[b], ps_v.at[b], qs_v.at[b], out_v.at[b]

        @pl.when(grp < _NGROUPS)
        def _():
            c = cev[0, :]
            def row_body(i, a):
                r = i * 4
                for u in range(4):
                    ob[r + u, :] = jnp.maximum(
                        eb[r + u, :] + pb[r + u, :] + qb[r + u, :] + c, 0.0)
                return a
            lax.fori_loop(0, _GROUP // 4, row_body, 0)

        @pl.when(grp >= _NGROUPS)
        def _():
            zrow = jnp.zeros((_DE,), F32)
            def row_body0(i, a):
                r = i * 4
                for u in range(4):
                    ob[r + u, :] = zrow
                return a
            lax.fori_loop(0, _GROUP // 4, row_body0, 0)

        @pl.when(grp < _NGROUPS)
        def _():
            pltpu.sync_copy(ob, eout_hbm.at[pl.ds(base, _GROUP)])

        @pl.when(grp >= _NGROUPS)
        def _():
            pltpu.sync_copy(ob, trash_hbm)

        for k in range(_K):
            pltpu.sync_copy(ob.at[pl.ds(k * _CHUNK, _CHUNK)],
                            acc_sh.at[idxr_v.at[b, k]], add=True)

    pltpu.sync_copy(s_hbm.at[pl.ds(wid * _K, _K)], idxs_v.at[0])
    pltpu.sync_copy(r_hbm.at[pl.ds(wid * _K, _K)], idxr_v.at[0])
    pltpu.sync_copy(ce_hbm, cev)
    issue_main(0, 0)
    issue_idx(1, 1)

    zrow = jnp.zeros((_DE,), F32)
    def zero_body(i, a):
        r = i * 8
        for u in range(8):
            out_v[0, r + u, :] = zrow
        return a
    lax.fori_loop(0, _GROUP // 8, zero_body, 0)
    pltpu.sync_copy(out_v.at[0], acc_sh.at[pl.ds(row0, _GROUP)])
    pltpu.sync_copy(out_v.at[0].at[pl.ds(0, _ROWS_PER_SUB - _GROUP)],
                    acc_sh.at[pl.ds(row0 + _GROUP, _ROWS_PER_SUB - _GROUP)])
    plsc.subcore_barrier()

    def steady(jj, carry):
        for b in (0, 1):
            j = 2 * jj + b
            wait_idx(j + 1, 1 - b)
            wait_main(j, b)
            issue_main(j + 1, 1 - b)
            finish(j, b)
            issue_idx(j + 2, b)
        return carry

    lax.fori_loop(0, _NJ // 2 - 1, steady, 0)

    wait_idx(_NJ - 1, 1)
    wait_main(_NJ - 2, 0)
    issue_main(_NJ - 1, 1)
    finish(_NJ - 2, 0)
    wait_main(_NJ - 1, 1)
    finish(_NJ - 1, 1)

    plsc.subcore_barrier()
    pltpu.sync_copy(acc_sh.at[pl.ds(row0, _ROWS_PER_SUB)],
                    recv_hbm.at[cid, pl.ds(row0, _ROWS_PER_SUB)])


def _node_body(nodes_ref, recv2_ref, g_ref,
               wnn_ref, wnr_ref, wng_ref, wnb_ref,
               wes_ref, wer_ref, weg_ref, web_ref,
               wgn_ref, wge_ref, wgg_ref, wgb_ref,
               nout_ref, p_ref, q_ref, gout_ref, ce_ref,
               accn_ref, acce_ref):
    i = pl.program_id(0)
    g = g_ref[...]
    c_n = jnp.dot(g, wng_ref[...], preferred_element_type=F32) + wnb_ref[...]
    recv = recv2_ref[0] + recv2_ref[1]
    out = (
        jnp.dot(nodes_ref[...], wnn_ref[...], preferred_element_type=F32)
        + jnp.dot(recv, wnr_ref[...], preferred_element_type=F32)
        + c_n
    )
    out = jnp.maximum(out, 0.0)
    nout_ref[...] = out
    p_ref[...] = jnp.dot(out, wes_ref[...], preferred_element_type=F32)
    q_ref[...] = jnp.dot(out, wer_ref[...], preferred_element_type=F32)

    @pl.when(i == 0)
    def _():
        accn_ref[...] = jnp.zeros_like(accn_ref)
        acce_ref[...] = jnp.zeros_like(acce_ref)

    accn_ref[...] += jnp.sum(out, axis=0, keepdims=True)
    acce_ref[...] += jnp.sum(recv, axis=0, keepdims=True)

    @pl.when(i == pl.num_programs(0) - 1)
    def _():
        agg_n = accn_ref[...]
        agg_e = acce_ref[...]
        g_new = (
            jnp.dot(agg_n, wgn_ref[...], preferred_element_type=F32)
            + jnp.dot(agg_e, wge_ref[...], preferred_element_type=F32)
            + jnp.dot(g, wgg_ref[...], preferred_element_type=F32)
            + wgb_ref[...]
        )
        gout_ref[...] = g_new
        ce_ref[...] = (
            jnp.dot(g_new, weg_ref[...], preferred_element_type=F32)
            + web_ref[...]
        )


def _node_fused_body(nodes_ref, recv2_ref, g_ref, edges_ref,
                     wnn_ref, wnr_ref, wng_ref, wnb_ref,
                     wes_ref, wer_ref, weg_ref, web_ref,
                     wgn_ref, wge_ref, wgg_ref, wgb_ref, wbd_ref,
                     nout_ref, p_ref, q_ref, gout_ref, ce_ref, ee_ref,
                     accn_ref, acce_ref):
    _node_body(nodes_ref, recv2_ref, g_ref,
               wnn_ref, wnr_ref, wng_ref, wnb_ref,
               wes_ref, wer_ref, weg_ref, web_ref,
               wgn_ref, wge_ref, wgg_ref, wgb_ref,
               nout_ref, p_ref, q_ref, gout_ref, ce_ref,
               accn_ref, acce_ref)
    ee_ref[...] = jnp.dot(edges_ref[...], wbd_ref[...],
                          preferred_element_type=F32)


def _full(i):
    return 0


def kernel(nodes, edges, globals_, senders, receivers,
           We_W, We_b, Wn_W, Wn_b, Wg_W, Wg_b):
    We_e = We_W[:_DE]
    We_s = We_W[_DE:_DE + _DN]
    We_r = We_W[_DE + _DN:_DE + 2 * _DN]
    We_g = We_W[_DE + 2 * _DN:]
    Wn_n = Wn_W[:_DN]
    Wn_r = Wn_W[_DN:_DN + _DE]
    Wn_g = Wn_W[_DN + _DE:]
    Wg_n = Wg_W[:_DN]
    Wg_e = Wg_W[_DN:_DN + _DE]
    Wg_g = Wg_W[_DN + _DE:]
    web = We_b.reshape(1, _DE)
    wnb = Wn_b.reshape(1, _DN)
    wgb = Wg_b.reshape(1, _DG)
    idx_pad = jnp.zeros((_EPAD - _E,), senders.dtype)
    spad = jnp.concatenate([senders, idx_pad]).reshape(_NCHPAD, _CHUNK)
    rpad = jnp.concatenate([receivers, idx_pad]).reshape(_NCHPAD, _CHUNK)

    n_grid = _N // _NBLK

    W_bd = jnp.kron(jnp.eye(8, dtype=F32), We_e)
    edges_p0 = edges.reshape(_E8, 128)

    p0, q0, ce0, ee0 = pl.pallas_call(
        _init_body,
        grid=(n_grid,),
        in_specs=[
            pl.BlockSpec((_NBLK, _DN), lambda i: (i, 0)),
            pl.BlockSpec((_EBLKF, 128), lambda i: (i, 0)),
            pl.BlockSpec((1, _DG), lambda i: (0, 0)),
            pl.BlockSpec((_DN, _DE), lambda i: (0, 0)),
            pl.BlockSpec((_DN, _DE), lambda i: (0, 0)),
            pl.BlockSpec((_DG, _DE), lambda i: (0, 0)),
            pl.BlockSpec((1, _DE), lambda i: (0, 0)),
            pl.BlockSpec((128, 128), lambda i: (0, 0)),
        ],
        out_specs=[
            pl.BlockSpec((_NBLK, _DE), lambda i: (i, 0)),
            pl.BlockSpec((_NBLK, _DE), lambda i: (i, 0)),
            pl.BlockSpec((1, _DE), lambda i: (0, 0)),
            pl.BlockSpec((_EBLKF, 128), lambda i: (i, 0)),
        ],
        out_shape=[
            jax.ShapeDtypeStruct((_N, _DE), F32),
            jax.ShapeDtypeStruct((_N, _DE), F32),
            jax.ShapeDtypeStruct((1, _DE), F32),
            jax.ShapeDtypeStruct((_EPAD // 8, 128), F32),
        ],
    )(nodes, edges_p0, globals_, We_s, We_r, We_g, web, W_bd)

    sc_step = pl.kernel(
        _sc_step_body,
        out_type=[
            jax.ShapeDtypeStruct((_E, _DE), F32),
            jax.ShapeDtypeStruct((_GROUP, _DE), F32),
            jax.ShapeDtypeStruct((2, _NPAD, _DE), F32),
        ],
        mesh=plsc.VectorSubcoreMesh(core_axis_name="c", subcore_axis_name="s"),
        compiler_params=pltpu.CompilerParams(use_tc_tiling_on_sc=False),
        scratch_types=[
            pltpu.VMEM((2, _K, _CHUNK), jnp.int32),
            pltpu.VMEM((2, _K, _CHUNK), jnp.int32),
            pltpu.VMEM((2, _GROUP, _DE), F32),
            pltpu.VMEM((2, _GROUP, _DE), F32),
            pltpu.VMEM((2, _GROUP, _DE), F32),
            pltpu.VMEM((2, _GROUP, _DE), F32),
            pltpu.VMEM((1, _DE), F32),
            pltpu.VMEM_SHARED((_NPAD, _DE), F32),
            pltpu.SemaphoreType.DMA,
            pltpu.SemaphoreType.DMA,
            pltpu.SemaphoreType.DMA,
            pltpu.SemaphoreType.DMA,
            pltpu.SemaphoreType.DMA,
            pltpu.SemaphoreType.DMA,
        ],
    )

    node_step = pl.pallas_call(
        _node_body,
        grid=(n_grid,),
        in_specs=[
            pl.BlockSpec((_NBLK, _DN), lambda i: (i, 0)),
            pl.BlockSpec((2, _NBLK, _DE), lambda i: (0, i, 0)),
            pl.BlockSpec((1, _DG), lambda i: (0, 0)),
            pl.BlockSpec((_DN, _DN), lambda i: (0, 0)),
            pl.BlockSpec((_DE, _DN), lambda i: (0, 0)),
            pl.BlockSpec((_DG, _DN), lambda i: (0, 0)),
            pl.BlockSpec((1, _DN), lambda i: (0, 0)),
            pl.BlockSpec((_DN, _DE), lambda i: (0, 0)),
            pl.BlockSpec((_DN, _DE), lambda i: (0, 0)),
            pl.BlockSpec((_DG, _DE), lambda i: (0, 0)),
            pl.BlockSpec((1, _DE), lambda i: (0, 0)),
            pl.BlockSpec((_DN, _DG), lambda i: (0, 0)),
            pl.BlockSpec((_DE, _DG), lambda i: (0, 0)),
            pl.BlockSpec((_DG, _DG), lambda i: (0, 0)),
            pl.BlockSpec((1, _DG), lambda i: (0, 0)),
        ],
        out_specs=[
            pl.BlockSpec((_NBLK, _DN), lambda i: (i, 0)),
            pl.BlockSpec((_NBLK, _DE), lambda i: (i, 0)),
            pl.BlockSpec((_NBLK, _DE), lambda i: (i, 0)),
            pl.BlockSpec((1, _DG), lambda i: (0, 0)),
            pl.BlockSpec((1, _DE), lambda i: (0, 0)),
        ],
        out_shape=[
            jax.ShapeDtypeStruct((_N, _DN), F32),
            jax.ShapeDtypeStruct((_N, _DE), F32),
            jax.ShapeDtypeStruct((_N, _DE), F32),
            jax.ShapeDtypeStruct((1, _DG), F32),
            jax.ShapeDtypeStruct((1, _DE), F32),
        ],
        scratch_shapes=[pltpu.VMEM((1, _DN), F32), pltpu.VMEM((1, _DE), F32)],
    )

    node_step_fused = pl.pallas_call(
        _node_fused_body,
        grid=(n_grid,),
        in_specs=[
            pl.BlockSpec((_NBLK, _DN), lambda i: (i, 0)),
            pl.BlockSpec((2, _NBLK, _DE), lambda i: (0, i, 0)),
            pl.BlockSpec((1, _DG), lambda i: (0, 0)),
            pl.BlockSpec((_EBLKF, 128), lambda i: (i, 0)),
            pl.BlockSpec((_DN, _DN), lambda i: (0, 0)),
            pl.BlockSpec((_DE, _DN), lambda i: (0, 0)),
            pl.BlockSpec((_DG, _DN), lambda i: (0, 0)),
            pl.BlockSpec((1, _DN), lambda i: (0, 0)),
            pl.BlockSpec((_DN, _DE), lambda i: (0, 0)),
            pl.BlockSpec((_DN, _DE), lambda i: (0, 0)),
            pl.BlockSpec((_DG, _DE), lambda i: (0, 0)),
            pl.BlockSpec((1, _DE), lambda i: (0, 0)),
            pl.BlockSpec((_DN, _DG), lambda i: (0, 0)),
            pl.BlockSpec((_DE, _DG), lambda i: (0, 0)),
            pl.BlockSpec((_DG, _DG), lambda i: (0, 0)),
            pl.BlockSpec((1, _DG), lambda i: (0, 0)),
            pl.BlockSpec((128, 128), lambda i: (0, 0)),
        ],
        out_specs=[
            pl.BlockSpec((_NBLK, _DN), lambda i: (i, 0)),
            pl.BlockSpec((_NBLK, _DE), lambda i: (i, 0)),
            pl.BlockSpec((_NBLK, _DE), lambda i: (i, 0)),
            pl.BlockSpec((1, _DG), lambda i: (0, 0)),
            pl.BlockSpec((1, _DE), lambda i: (0, 0)),
            pl.BlockSpec((_EBLKF, 128), lambda i: (i, 0)),
        ],
        out_shape=[
            jax.ShapeDtypeStruct((_N, _DN), F32),
            jax.ShapeDtypeStruct((_N, _DE), F32),
            jax.ShapeDtypeStruct((_N, _DE), F32),
            jax.ShapeDtypeStruct((1, _DG), F32),
            jax.ShapeDtypeStruct((1, _DE), F32),
            jax.ShapeDtypeStruct((_EPAD // 8, 128), F32),
        ],
        scratch_shapes=[pltpu.VMEM((1, _DN), F32), pltpu.VMEM((1, _DE), F32)],
    )

    p, q, ce, g, ee_p = p0, q0, ce0, globals_, ee0
    for _step in range(3):
        edges_lin, _trash, recv2 = sc_step(
            ee_p.reshape(_EPAD, _DE), p, q, spad, rpad, ce)
        edges_p = edges_lin.reshape(_E8, 128)
        if _step < 2:
            nodes, p, q, g, ce, ee_p = node_step_fused(
                nodes, recv2, g, edges_p,
                Wn_n, Wn_r, Wn_g, wnb,
                We_s, We_r, We_g, web,
                Wg_n, Wg_e, Wg_g, wgb, W_bd,
            )
        else:
            nodes, p, q, g, ce = node_step(
                nodes, recv2, g,
                Wn_n, Wn_r, Wn_g, wnb,
                We_s, We_r, We_g, web,
                Wg_n, Wg_e, Wg_g, wgb,
            )

    return (nodes, edges_lin, g)
